# baseline (speedup 1.0000x reference)
import functools

import jax
import jax.numpy as jnp
from jax import lax
from jax.experimental import pallas as pl
from jax.experimental.pallas import tpu as pltpu
from jax.experimental.pallas import tpu_sc as plsc

NUM_EMBEDDINGS = 1000000
EMBEDDING_DIM = 64
BATCH_SIZE = 16384

_HALF_V = NUM_EMBEDDINGS // 2
_NUM_CORES = 2
_NUM_SUBCORES = 16
_NUM_WORKERS = _NUM_CORES * _NUM_SUBCORES
_B_PER_W = BATCH_SIZE // _NUM_WORKERS
_CHUNK = 128
_NCHUNK = _B_PER_W // _CHUNK
_HB = 256

_MESH = plsc.VectorSubcoreMesh(core_axis_name="c", subcore_axis_name="s")


@functools.partial(
    pl.kernel,
    mesh=_MESH,
    compiler_params=pltpu.CompilerParams(use_tc_tiling_on_sc=False),
    out_type=jax.ShapeDtypeStruct((BATCH_SIZE, EMBEDDING_DIM), jnp.float32),
    scratch_types=[
        pltpu.VMEM((_B_PER_W,), jnp.int32),
        pltpu.VMEM((_NCHUNK, _CHUNK), jnp.int32),
        pltpu.VMEM((_NCHUNK, _CHUNK), jnp.int32),
        pltpu.VMEM((_HB, EMBEDDING_DIM), jnp.float32),
        pltpu.VMEM((_HB, EMBEDDING_DIM), jnp.float32),
        pltpu.SemaphoreType.DMA,
        pltpu.SemaphoreType.DMA,
    ],
)
def _sc_gather(idx_hbm, idx0_hbm, idx1_hbm, t0_hbm, t1_hbm, out_hbm,
               idx_v, idx0_v, idx1_v, rows_a, rows_b, sem_a, sem_b):
    wid = lax.axis_index("s") * _NUM_CORES + lax.axis_index("c")
    base = wid * _B_PER_W

    pltpu.sync_copy(idx_hbm.at[wid], idx_v)
    pltpu.sync_copy(idx0_hbm.at[wid], idx0_v)
    pltpu.sync_copy(idx1_hbm.at[wid], idx1_v)

    def half_batch(hb):
        waits = []
        for sub in range(_HB // _CHUNK):
            j = hb * (_HB // _CHUNK) + sub
            dst = pl.ds(sub * _CHUNK, _CHUNK)
            waits.append(
                pltpu.async_copy(t0_hbm.at[idx0_v.at[j]],
                                 rows_a.at[dst], sem_a)
            )
            waits.append(
                pltpu.async_copy(t1_hbm.at[idx1_v.at[j]],
                                 rows_b.at[dst], sem_b)
            )
        for w in waits:
            w.wait()

        for k in range(_HB):
            vg = idx_v[pl.ds(hb * _HB + (k // 16) * 16, 16)]

            @pl.when(vg[k % 16] >= _HALF_V)
            def _():
                for c in range(EMBEDDING_DIM // 16):
                    cs = pl.ds(c * 16, 16)
                    rows_a.at[k][cs] = rows_b.at[k][cs]

        pltpu.sync_copy(rows_a, out_hbm.at[pl.ds(base + hb * _HB, _HB)])

    pl.loop(0, _B_PER_W // _HB)(half_batch)


def kernel(inputs, embeddings):
    idx = inputs.astype(jnp.int32)
    t0 = embeddings[:_HALF_V]
    t1 = embeddings[_HALF_V:]
    idx0 = jnp.minimum(idx, _HALF_V - 1)
    idx1 = jnp.maximum(idx, _HALF_V) - _HALF_V
    shape = (_NUM_WORKERS, _NCHUNK, _CHUNK)
    return _sc_gather(
        idx.reshape(_NUM_WORKERS, _B_PER_W),
        idx0.reshape(shape),
        idx1.reshape(shape),
        t0,
        t1,
    )


# speedup vs baseline: 1.7595x; 1.7595x over previous
import functools

import jax
import jax.numpy as jnp
from jax import lax
from jax.experimental import pallas as pl
from jax.experimental.pallas import tpu as pltpu
from jax.experimental.pallas import tpu_sc as plsc

NUM_EMBEDDINGS = 1000000
EMBEDDING_DIM = 64
BATCH_SIZE = 16384

_NUM_CORES = 2
_NUM_SUBCORES = 16
_NUM_WORKERS = _NUM_CORES * _NUM_SUBCORES
_B_PER_W = BATCH_SIZE // _NUM_WORKERS

_MESH = plsc.VectorSubcoreMesh(core_axis_name="c", subcore_axis_name="s")


@functools.partial(
    pl.kernel,
    mesh=_MESH,
    out_type=jax.ShapeDtypeStruct((BATCH_SIZE, EMBEDDING_DIM), jnp.float32),
    scratch_types=[
        pltpu.VMEM((_B_PER_W,), jnp.int32),
        pltpu.SemaphoreType.DMA,
    ],
)
def _sc_gather(idx_hbm, table_hbm, out_hbm, idx_v, sem):
    wid = lax.axis_index("s") * _NUM_CORES + lax.axis_index("c")
    base = wid * _B_PER_W

    pltpu.sync_copy(idx_hbm.at[wid], idx_v)

    def fire(g):
        vg = idx_v[pl.ds(g * 16, 16)]
        for l in range(16):
            pltpu.async_copy(
                table_hbm.at[pl.ds(vg[l], 1)],
                out_hbm.at[pl.ds(base + g * 16 + l, 1)],
                sem,
            )

    pl.loop(0, _B_PER_W // 16)(fire)

    pltpu.make_async_copy(
        table_hbm.at[pl.ds(0, _B_PER_W)],
        out_hbm.at[pl.ds(base, _B_PER_W)],
        sem,
    ).wait()


def kernel(inputs, embeddings):
    idx = inputs.astype(jnp.int32).reshape(_NUM_WORKERS, _B_PER_W)
    return _sc_gather(idx, embeddings)


# speedup vs baseline: 2.3527x; 1.3372x over previous
import functools

import jax
import jax.numpy as jnp
from jax import lax
from jax.experimental import pallas as pl
from jax.experimental.pallas import tpu as pltpu
from jax.experimental.pallas import tpu_sc as plsc

NUM_EMBEDDINGS = 1000000
EMBEDDING_DIM = 64
BATCH_SIZE = 16384

_NUM_CORES = 2
_NUM_SUBCORES = 16
_NUM_WORKERS = _NUM_CORES * _NUM_SUBCORES
_B_PER_W = BATCH_SIZE // _NUM_WORKERS
_N_STREAM = 320
_N_DMA = _B_PER_W - _N_STREAM

_MESH = plsc.VectorSubcoreMesh(core_axis_name="c", subcore_axis_name="s")


@functools.partial(
    pl.kernel,
    mesh=_MESH,
    out_type=jax.ShapeDtypeStruct((BATCH_SIZE, EMBEDDING_DIM), jnp.float32),
    scratch_types=[
        pltpu.VMEM((_B_PER_W,), jnp.int32),
        pltpu.VMEM((_N_STREAM, EMBEDDING_DIM), jnp.float32),
        pltpu.SemaphoreType.DMA,
        pltpu.SemaphoreType.DMA,
    ],
)
def _sc_gather(idx_hbm, table_hbm, out_hbm, idx_v, rows_v, sem, dsem):
    wid = lax.axis_index("s") * _NUM_CORES + lax.axis_index("c")
    base = wid * _B_PER_W

    pltpu.sync_copy(idx_hbm.at[wid], idx_v)

    def fire(g):
        vg = idx_v[pl.ds(g * 16, 16)]
        i0 = g * 16
        if g < _N_STREAM // 16:
            for l in range(16):
                pltpu.async_copy(
                    table_hbm.at[pl.ds(vg[l], 1)],
                    rows_v.at[pl.ds(i0 + l, 1)],
                    sem,
                )
        else:
            for l in range(16):
                pltpu.async_copy(
                    table_hbm.at[pl.ds(vg[l], 1)],
                    out_hbm.at[pl.ds(base + i0 + l, 1)],
                    dsem,
                )

    order = []
    s, d = 0, _N_STREAM // 16
    while s < _N_STREAM // 16 or d < _B_PER_W // 16:
        if s < _N_STREAM // 16:
            order.append(s)
            s += 1
        if s < _N_STREAM // 16:
            order.append(s)
            s += 1
        if d < _B_PER_W // 16:
            order.append(d)
            d += 1
    for g in order:
        fire(g)

    pltpu.make_async_copy(
        table_hbm.at[pl.ds(0, _N_STREAM)], rows_v, sem
    ).wait()
    pltpu.sync_copy(rows_v, out_hbm.at[pl.ds(base, _N_STREAM)])
    pltpu.make_async_copy(
        table_hbm.at[pl.ds(0, _N_DMA)],
        out_hbm.at[pl.ds(base + _N_STREAM, _N_DMA)],
        dsem,
    ).wait()


def kernel(inputs, embeddings):
    idx = inputs.astype(jnp.int32).reshape(_NUM_WORKERS, _B_PER_W)
    return _sc_gather(idx, embeddings)


# speedup vs baseline: 2.9432x; 1.2510x over previous
import functools

import jax
import jax.numpy as jnp
from jax import lax
from jax.experimental import pallas as pl
from jax.experimental.pallas import tpu as pltpu
from jax.experimental.pallas import tpu_sc as plsc

NUM_EMBEDDINGS = 1000000
EMBEDDING_DIM = 64
BATCH_SIZE = 16384

_NUM_CORES = 2
_NUM_SUBCORES = 16
_NUM_WORKERS = _NUM_CORES * _NUM_SUBCORES
_B_PER_W = BATCH_SIZE // _NUM_WORKERS

_MESH = plsc.VectorSubcoreMesh(core_axis_name="c", subcore_axis_name="s")


@functools.partial(
    pl.kernel,
    mesh=_MESH,
    out_type=jax.ShapeDtypeStruct((BATCH_SIZE, EMBEDDING_DIM), jnp.float32),
    scratch_types=[
        pltpu.VMEM((_B_PER_W,), jnp.int32),
        pltpu.VMEM((_B_PER_W, EMBEDDING_DIM), jnp.float32),
        pltpu.SemaphoreType.DMA,
    ],
)
def _sc_gather(idx_hbm, table_hbm, out_hbm, idx_v, rows_v, sem):
    wid = lax.axis_index("s") * _NUM_CORES + lax.axis_index("c")
    base = wid * _B_PER_W

    pltpu.sync_copy(idx_hbm.at[wid], idx_v)

    def fire(g):
        vg = idx_v[pl.ds(g * 16, 16)]
        for l in range(16):
            row = vg[l]
            pltpu.async_copy(
                table_hbm.at[pl.ds(row, 1)],
                rows_v.at[pl.ds(g * 16 + l, 1)],
                sem,
            )

    pl.loop(0, _B_PER_W // 16)(fire)

    pltpu.make_async_copy(table_hbm.at[pl.ds(0, _B_PER_W)], rows_v, sem).wait()
    pltpu.sync_copy(rows_v, out_hbm.at[pl.ds(base, _B_PER_W)])


def kernel(inputs, embeddings):
    idx = inputs.astype(jnp.int32).reshape(_NUM_WORKERS, _B_PER_W)
    return _sc_gather(idx, embeddings)
